# baseline (speedup 1.0000x reference)
import functools

import jax
import jax.numpy as jnp
from jax import lax
from jax.experimental import pallas as pl
from jax.experimental.pallas import tpu as pltpu
from jax.experimental.pallas import tpu_sc as plsc

N = 10000
D = 128
E = 320000

NC = 2
NS = 16
NW = NC * NS
L = 16

EB = 128
NBLK = 80
CHUNK = 16
BPT = NBLK * EB
E_PAD = NW * BPT
NPAD = 10240
DEG_STRIPE = NPAD // NS
ROW_STRIPE = NPAD // NS
ZROWS = 16

MB = 2048
GRID = NPAD // MB

_MESH = plsc.VectorSubcoreMesh(core_axis_name="c", subcore_axis_name="s")


@functools.partial(
    pl.kernel,
    out_type=jax.ShapeDtypeStruct((NC, 1, NPAD), jnp.float32),
    mesh=_MESH,
    scratch_types=[
        pltpu.VMEM((NBLK, EB), jnp.int32),
        pltpu.VMEM((NBLK, EB), jnp.float32),
        pltpu.VMEM((DEG_STRIPE,), jnp.float32),
        pltpu.VMEM_SHARED((NPAD,), jnp.float32),
        pltpu.SemaphoreType.DMA,
    ],
)
def _deg_kernel(dst_hbm, w_hbm, zero_hbm, out_hbm, idx_v, w_v, z_v, acc_s, sem):
    c = lax.axis_index("c")
    s = lax.axis_index("s")
    wid = c * NS + s
    pltpu.sync_copy(zero_hbm, z_v)
    pltpu.sync_copy(z_v, acc_s.at[pl.ds(s * DEG_STRIPE, DEG_STRIPE)])
    plsc.subcore_barrier()
    pltpu.sync_copy(dst_hbm.at[pl.ds(wid * NBLK, NBLK)], idx_v)
    pltpu.sync_copy(w_hbm.at[pl.ds(wid * NBLK, NBLK)], w_v)

    @pl.loop(0, NBLK, step=8)
    def _(k):
        for j in range(8):
            pltpu.async_copy(w_v.at[k + j], acc_s.at[idx_v.at[k + j]], sem,
                             add=True)
        for j in range(8):
            pltpu.make_async_copy(w_v.at[k + j], acc_s.at[idx_v.at[k + j]],
                                  sem).wait()

    plsc.subcore_barrier()
    pltpu.sync_copy(acc_s.at[pl.ds(s * DEG_STRIPE, DEG_STRIPE)],
                    out_hbm.at[c, 0, pl.ds(s * DEG_STRIPE, DEG_STRIPE)])


@functools.partial(
    pl.kernel,
    out_type=jax.ShapeDtypeStruct((NC, NPAD, D), jnp.float32),
    mesh=_MESH,
    scratch_types=[
        pltpu.VMEM((NBLK, EB), jnp.int32),
        pltpu.VMEM((CHUNK, EB), jnp.int32),
        pltpu.VMEM((CHUNK, EB), jnp.float32),
        pltpu.VMEM((EB, D), jnp.float32),
        pltpu.VMEM((EB, D), jnp.float32),
        pltpu.VMEM_SHARED((N, D), jnp.float32),
        pltpu.SemaphoreType.DMA,
        pltpu.SemaphoreType.DMA,
    ],
)
def _agg_kernel(h2_hbm, src_hbm, dst_hbm, w_hbm, zero_hbm, out_hbm,
                src_v, dst_c, w_c, rows0, rows1, acc_s, g0, g1):
    c = lax.axis_index("c")
    s = lax.axis_index("s")
    wid = c * NS + s
    pltpu.sync_copy(zero_hbm, rows0.at[pl.ds(0, ZROWS)])

    @pl.loop(0, ROW_STRIPE // ZROWS)
    def _(r):
        base = s * ROW_STRIPE + r * ZROWS

        @pl.when(base < N)
        def _():
            pltpu.sync_copy(rows0.at[pl.ds(0, ZROWS)],
                            acc_s.at[pl.ds(base, ZROWS)])

    plsc.subcore_barrier()
    pltpu.sync_copy(src_hbm.at[pl.ds(wid * NBLK, NBLK)], src_v)

    def process(k, buf, gsem):
        kk = lax.rem(k, CHUNK)
        pltpu.make_async_copy(h2_hbm.at[src_v.at[k]], buf, gsem).wait()

        @pl.loop(0, EB // L)
        def _(g):
            w16 = w_c[kk, pl.ds(g * L, L)]
            for j in range(L):
                sw = jnp.take_along_axis(
                    w16, jnp.full((L,), j, jnp.int32), axis=0)
                e = g * L + j
                for i in range(D // L):
                    sl = pl.ds(i * L, L)
                    buf[e, sl] = buf[e, sl] * sw

        pltpu.sync_copy(buf, acc_s.at[dst_c.at[kk]], add=True)

    pltpu.async_copy(h2_hbm.at[src_v.at[0]], rows0, g0)
    pltpu.async_copy(h2_hbm.at[src_v.at[1]], rows1, g1)

    @pl.loop(0, NBLK, step=2)
    def _(k):
        @pl.when(lax.rem(k, CHUNK) == 0)
        def _():
            kc = pl.multiple_of(wid * NBLK + k, CHUNK)
            pltpu.sync_copy(dst_hbm.at[pl.ds(kc, CHUNK)], dst_c)
            pltpu.sync_copy(w_hbm.at[pl.ds(kc, CHUNK)], w_c)

        process(k, rows0, g0)

        @pl.when(k + 2 < NBLK)
        def _():
            pltpu.async_copy(h2_hbm.at[src_v.at[k + 2]], rows0, g0)

        process(k + 1, rows1, g1)

        @pl.when(k + 3 < NBLK)
        def _():
            pltpu.async_copy(h2_hbm.at[src_v.at[k + 3]], rows1, g1)

    plsc.subcore_barrier()

    @pl.loop(0, ROW_STRIPE // ZROWS)
    def _(r):
        base = s * ROW_STRIPE + r * ZROWS

        @pl.when(base < N)
        def _():
            pltpu.sync_copy(acc_s.at[pl.ds(base, ZROWS)],
                            out_hbm.at[c, pl.ds(base, ZROWS)])


def _dis_from(dp_ref):
    deg = dp_ref[0, 0, :] + dp_ref[1, 0, :]
    return jnp.where(deg > 0, 1.0 / jnp.sqrt(jnp.maximum(deg, 1e-12)), 0.0)


_DEG_SPEC = pl.BlockSpec((NC, 1, MB), lambda i: (0, 0, i))


def _mm_body(x_ref, w_ref, o_ref):
    o_ref[...] = jnp.dot(x_ref[...], w_ref[...],
                         preferred_element_type=jnp.float32)


_mm_call = pl.pallas_call(
    _mm_body,
    grid=(GRID,),
    in_specs=[
        pl.BlockSpec((MB, D), lambda i: (i, 0)),
        pl.BlockSpec((D, D), lambda i: (0, 0)),
    ],
    out_specs=pl.BlockSpec((MB, D), lambda i: (i, 0)),
    out_shape=jax.ShapeDtypeStruct((NPAD, D), jnp.float32),
)


def _h2_body(h_ref, dp_ref, o_ref):
    dis = _dis_from(dp_ref)
    o_ref[...] = h_ref[...] * dis[:, None]


_h2_call = pl.pallas_call(
    _h2_body,
    grid=(GRID,),
    in_specs=[
        pl.BlockSpec((MB, D), lambda i: (i, 0)),
        _DEG_SPEC,
    ],
    out_specs=pl.BlockSpec((MB, D), lambda i: (i, 0)),
    out_shape=jax.ShapeDtypeStruct((NPAD, D), jnp.float32),
)


def _fin_body(p_ref, dp_ref, b_ref, o_ref):
    dis = _dis_from(dp_ref)
    z = (p_ref[0] + p_ref[1]) * dis[:, None] + b_ref[...]
    o_ref[...] = jnp.where(z > 0, z, 0.1 * (jnp.exp(z) - 1.0))


_fin_call = pl.pallas_call(
    _fin_body,
    grid=(GRID,),
    in_specs=[
        pl.BlockSpec((NC, MB, D), lambda i: (0, i, 0)),
        _DEG_SPEC,
        pl.BlockSpec((D,), lambda i: (0,)),
    ],
    out_specs=pl.BlockSpec((MB, D), lambda i: (i, 0)),
    out_shape=jax.ShapeDtypeStruct((NPAD, D), jnp.float32),
)


def kernel(x, edge_index, edge_attrs, W, b):
    src = edge_index[0]
    dst = edge_index[1]
    pad = E_PAD - E
    pad_idx = jnp.arange(pad, dtype=jnp.int32) & 8191
    src_p = jnp.concatenate([src, pad_idx]).reshape(NW * NBLK, EB)
    dst_p = jnp.concatenate([dst, pad_idx]).reshape(NW * NBLK, EB)
    w_p = jnp.concatenate(
        [edge_attrs, jnp.zeros((pad,), jnp.float32)]).reshape(NW * NBLK, EB)

    x_pad = jnp.concatenate(
        [x, jnp.zeros((NPAD - N, D), jnp.float32)], axis=0)
    h_raw = _mm_call(x_pad, W)
    degp = _deg_kernel(dst_p, w_p, jnp.zeros((DEG_STRIPE,), jnp.float32))
    h2 = _h2_call(h_raw, degp)
    outp = _agg_kernel(h2, src_p, dst_p, w_p,
                       jnp.zeros((ZROWS, D), jnp.float32))
    return _fin_call(outp, degp, b)[:N]


# speedup vs baseline: 1.0508x; 1.0508x over previous
import functools

import jax
import jax.numpy as jnp
from jax import lax
from jax.experimental import pallas as pl
from jax.experimental.pallas import tpu as pltpu
from jax.experimental.pallas import tpu_sc as plsc

N = 10000
D = 128
E = 320000

NC = 2
NS = 16
NW = NC * NS
L = 16

EB = 128
NBLK = 80
CHUNK = 16
BPT = NBLK * EB
E_PAD = NW * BPT
NPAD = 10240
DEG_STRIPE = NPAD // NS
ROW_STRIPE = NPAD // NS
ZROWS = 16

MB = 2048
GRID = NPAD // MB

_MESH = plsc.VectorSubcoreMesh(core_axis_name="c", subcore_axis_name="s")


@functools.partial(
    pl.kernel,
    out_type=jax.ShapeDtypeStruct((NC, 1, NPAD), jnp.float32),
    mesh=_MESH,
    scratch_types=[
        pltpu.VMEM((NBLK, EB), jnp.int32),
        pltpu.VMEM((NBLK, EB), jnp.float32),
        pltpu.VMEM((DEG_STRIPE,), jnp.float32),
        pltpu.VMEM_SHARED((NPAD,), jnp.float32),
        pltpu.SemaphoreType.DMA,
    ],
)
def _deg_kernel(ei_hbm, w_hbm, zero_hbm, out_hbm, idx_v, w_v, z_v, acc_s, sem):
    c = lax.axis_index("c")
    s = lax.axis_index("s")
    wid = c * NS + s
    pltpu.sync_copy(zero_hbm, z_v)
    pltpu.sync_copy(z_v, acc_s.at[pl.ds(s * DEG_STRIPE, DEG_STRIPE)])
    plsc.subcore_barrier()
    pltpu.sync_copy(ei_hbm.at[1, pl.ds(wid * NBLK, NBLK)], idx_v)
    pltpu.sync_copy(w_hbm.at[pl.ds(wid * NBLK, NBLK)], w_v)

    @pl.loop(0, NBLK, step=8)
    def _(k):
        for j in range(8):
            pltpu.async_copy(w_v.at[k + j], acc_s.at[idx_v.at[k + j]], sem,
                             add=True)
        for j in range(8):
            pltpu.make_async_copy(w_v.at[k + j], acc_s.at[idx_v.at[k + j]],
                                  sem).wait()

    plsc.subcore_barrier()
    pltpu.sync_copy(acc_s.at[pl.ds(s * DEG_STRIPE, DEG_STRIPE)],
                    out_hbm.at[c, 0, pl.ds(s * DEG_STRIPE, DEG_STRIPE)])


@functools.partial(
    pl.kernel,
    out_type=jax.ShapeDtypeStruct((NC, NPAD, D), jnp.float32),
    mesh=_MESH,
    scratch_types=[
        pltpu.VMEM((NBLK, EB), jnp.int32),
        pltpu.VMEM((CHUNK, EB), jnp.int32),
        pltpu.VMEM((CHUNK, EB), jnp.float32),
        pltpu.VMEM((EB, D), jnp.float32),
        pltpu.VMEM((EB, D), jnp.float32),
        pltpu.VMEM_SHARED((N, D), jnp.float32),
        pltpu.SemaphoreType.DMA,
        pltpu.SemaphoreType.DMA,
    ],
)
def _agg_kernel(h2_hbm, ei_hbm, w_hbm, zero_hbm, out_hbm,
                src_v, dst_c, w_c, rows0, rows1, acc_s, g0, g1):
    c = lax.axis_index("c")
    s = lax.axis_index("s")
    wid = c * NS + s
    pltpu.sync_copy(zero_hbm, rows0.at[pl.ds(0, ZROWS)])

    @pl.loop(0, ROW_STRIPE // ZROWS)
    def _(r):
        base = s * ROW_STRIPE + r * ZROWS

        @pl.when(base < N)
        def _():
            pltpu.sync_copy(rows0.at[pl.ds(0, ZROWS)],
                            acc_s.at[pl.ds(base, ZROWS)])

    plsc.subcore_barrier()
    pltpu.sync_copy(ei_hbm.at[0, pl.ds(wid * NBLK, NBLK)], src_v)

    def process(k, buf, gsem):
        kk = lax.rem(k, CHUNK)
        pltpu.make_async_copy(h2_hbm.at[src_v.at[k]], buf, gsem).wait()

        @pl.loop(0, EB // L)
        def _(g):
            w16 = w_c[kk, pl.ds(g * L, L)]
            for j in range(L):
                sw = jnp.take_along_axis(
                    w16, jnp.full((L,), j, jnp.int32), axis=0)
                e = g * L + j
                for i in range(D // L):
                    sl = pl.ds(i * L, L)
                    buf[e, sl] = buf[e, sl] * sw

        pltpu.sync_copy(buf, acc_s.at[dst_c.at[kk]], add=True)

    pltpu.async_copy(h2_hbm.at[src_v.at[0]], rows0, g0)
    pltpu.async_copy(h2_hbm.at[src_v.at[1]], rows1, g1)

    @pl.loop(0, NBLK, step=2)
    def _(k):
        @pl.when(lax.rem(k, CHUNK) == 0)
        def _():
            kc = pl.multiple_of(wid * NBLK + k, CHUNK)
            pltpu.sync_copy(ei_hbm.at[1, pl.ds(kc, CHUNK)], dst_c)
            pltpu.sync_copy(w_hbm.at[pl.ds(kc, CHUNK)], w_c)

        process(k, rows0, g0)

        @pl.when(k + 2 < NBLK)
        def _():
            pltpu.async_copy(h2_hbm.at[src_v.at[k + 2]], rows0, g0)

        process(k + 1, rows1, g1)

        @pl.when(k + 3 < NBLK)
        def _():
            pltpu.async_copy(h2_hbm.at[src_v.at[k + 3]], rows1, g1)

    plsc.subcore_barrier()

    @pl.loop(0, ROW_STRIPE // ZROWS)
    def _(r):
        base = s * ROW_STRIPE + r * ZROWS

        @pl.when(base < N)
        def _():
            pltpu.sync_copy(acc_s.at[pl.ds(base, ZROWS)],
                            out_hbm.at[c, pl.ds(base, ZROWS)])


def _dis_from(dp_ref):
    deg = dp_ref[0, 0, :] + dp_ref[1, 0, :]
    return jnp.where(deg > 0, 1.0 / jnp.sqrt(jnp.maximum(deg, 1e-12)), 0.0)


_DEG_SPEC = pl.BlockSpec((NC, 1, MB), lambda i: (0, 0, i))


def _mm_body(x_ref, w_ref, o_ref):
    o_ref[...] = jnp.dot(x_ref[...], w_ref[...],
                         preferred_element_type=jnp.float32)


_mm_call = pl.pallas_call(
    _mm_body,
    grid=(GRID,),
    in_specs=[
        pl.BlockSpec((MB, D), lambda i: (i, 0)),
        pl.BlockSpec((D, D), lambda i: (0, 0)),
    ],
    out_specs=pl.BlockSpec((MB, D), lambda i: (i, 0)),
    out_shape=jax.ShapeDtypeStruct((NPAD, D), jnp.float32),
)


def _h2_body(h_ref, dp_ref, o_ref):
    dis = _dis_from(dp_ref)
    o_ref[...] = h_ref[...] * dis[:, None]


_h2_call = pl.pallas_call(
    _h2_body,
    grid=(GRID,),
    in_specs=[
        pl.BlockSpec((MB, D), lambda i: (i, 0)),
        _DEG_SPEC,
    ],
    out_specs=pl.BlockSpec((MB, D), lambda i: (i, 0)),
    out_shape=jax.ShapeDtypeStruct((NPAD, D), jnp.float32),
)


def _fin_body(p_ref, dp_ref, b_ref, o_ref):
    dis = _dis_from(dp_ref)
    z = (p_ref[0] + p_ref[1]) * dis[:, None] + b_ref[...]
    o_ref[...] = jnp.where(z > 0, z, 0.1 * (jnp.exp(z) - 1.0))


_fin_call = pl.pallas_call(
    _fin_body,
    grid=(GRID,),
    in_specs=[
        pl.BlockSpec((NC, MB, D), lambda i: (0, i, 0)),
        _DEG_SPEC,
        pl.BlockSpec((D,), lambda i: (0,)),
    ],
    out_specs=pl.BlockSpec((MB, D), lambda i: (i, 0)),
    out_shape=jax.ShapeDtypeStruct((NPAD, D), jnp.float32),
)


def kernel(x, edge_index, edge_attrs, W, b):
    pad = E_PAD - E
    pad_idx = jnp.arange(pad, dtype=jnp.int32) & 8191
    ei_p = jnp.concatenate(
        [edge_index.reshape(2, E // EB, EB),
         jnp.broadcast_to(pad_idx.reshape(1, pad // EB, EB),
                          (2, pad // EB, EB))], axis=1)
    w_p = jnp.concatenate(
        [edge_attrs, jnp.zeros((pad,), jnp.float32)]).reshape(NW * NBLK, EB)

    x_pad = jnp.concatenate(
        [x, jnp.zeros((NPAD - N, D), jnp.float32)], axis=0)
    h_raw = _mm_call(x_pad, W)
    degp = _deg_kernel(ei_p, w_p, jnp.zeros((DEG_STRIPE,), jnp.float32))
    h2 = _h2_call(h_raw, degp)
    outp = _agg_kernel(h2, ei_p, w_p,
                       jnp.zeros((ZROWS, D), jnp.float32))
    return _fin_call(outp, degp, b)[:N]


# speedup vs baseline: 1.1574x; 1.1014x over previous
import functools

import jax
import jax.numpy as jnp
from jax import lax
from jax.experimental import pallas as pl
from jax.experimental.pallas import tpu as pltpu
from jax.experimental.pallas import tpu_sc as plsc

N = 10000
D = 128
E = 320000

NC = 2
NS = 16
NW = NC * NS
L = 16

EB = 128
NBLK = 80
CHUNK = 16
BPT = NBLK * EB
E_PAD = NW * BPT
NPAD = 10240
DEG_STRIPE = NPAD // NS
ROW_STRIPE = NPAD // NS
ZROWS = 16

MB = 2048
GRID = NPAD // MB

_MESH = plsc.VectorSubcoreMesh(core_axis_name="c", subcore_axis_name="s")


@functools.partial(
    pl.kernel,
    out_type=jax.ShapeDtypeStruct((NC, 1, NPAD), jnp.float32),
    mesh=_MESH,
    scratch_types=[
        pltpu.VMEM((NBLK, EB), jnp.int32),
        pltpu.VMEM((NBLK, EB), jnp.float32),
        pltpu.VMEM((DEG_STRIPE,), jnp.float32),
        pltpu.VMEM_SHARED((NPAD,), jnp.float32),
        pltpu.SemaphoreType.DMA,
    ],
)
def _deg_kernel(ei_hbm, w_hbm, zero_hbm, out_hbm, idx_v, w_v, z_v, acc_s, sem):
    c = lax.axis_index("c")
    s = lax.axis_index("s")
    wid = c * NS + s
    pltpu.sync_copy(zero_hbm, z_v)
    pltpu.sync_copy(z_v, acc_s.at[pl.ds(s * DEG_STRIPE, DEG_STRIPE)])
    plsc.subcore_barrier()
    pltpu.sync_copy(ei_hbm.at[1, pl.ds(wid * NBLK, NBLK)], idx_v)
    pltpu.sync_copy(w_hbm.at[pl.ds(wid * NBLK, NBLK)], w_v)

    @pl.loop(0, NBLK, step=8)
    def _(k):
        for j in range(8):
            pltpu.async_copy(w_v.at[k + j], acc_s.at[idx_v.at[k + j]], sem,
                             add=True)
        for j in range(8):
            pltpu.make_async_copy(w_v.at[k + j], acc_s.at[idx_v.at[k + j]],
                                  sem).wait()

    plsc.subcore_barrier()
    pltpu.sync_copy(acc_s.at[pl.ds(s * DEG_STRIPE, DEG_STRIPE)],
                    out_hbm.at[c, 0, pl.ds(s * DEG_STRIPE, DEG_STRIPE)])


@functools.partial(
    pl.kernel,
    out_type=jax.ShapeDtypeStruct((NC, NPAD, D), jnp.float32),
    mesh=_MESH,
    scratch_types=[
        pltpu.VMEM((NBLK, EB), jnp.int32),
        pltpu.VMEM((CHUNK, EB), jnp.int32),
        pltpu.VMEM((CHUNK, EB), jnp.float32),
        pltpu.VMEM((EB, D), jnp.float32),
        pltpu.VMEM((EB, D), jnp.float32),
        pltpu.VMEM_SHARED((N, D), jnp.float32),
        pltpu.SemaphoreType.DMA,
        pltpu.SemaphoreType.DMA,
    ],
)
def _agg_kernel(h2_hbm, ei_hbm, w_hbm, zero_hbm, out_hbm,
                src_v, dst_c, w_c, rows0, rows1, acc_s, g0, g1):
    c = lax.axis_index("c")
    s = lax.axis_index("s")
    wid = c * NS + s
    pltpu.sync_copy(zero_hbm, rows0)

    def _stripe_copy(copy_fn):
        base = s * ROW_STRIPE

        @pl.when(s < NS - 1)
        def _():
            @pl.loop(0, ROW_STRIPE // EB)
            def _(r):
                copy_fn(base + r * EB, EB)

        @pl.when(s == NS - 1)
        def _():
            @pl.loop(0, 3)
            def _(r):
                copy_fn(base + r * EB, EB)

            copy_fn(base + 3 * EB, ZROWS)

    def _zero_copy(row, nrows):
        pltpu.sync_copy(rows0.at[pl.ds(0, nrows)], acc_s.at[pl.ds(row, nrows)])

    _stripe_copy(_zero_copy)
    pltpu.sync_copy(ei_hbm.at[0, pl.ds(wid * NBLK, NBLK)], src_v)
    plsc.subcore_barrier()

    def process(k, buf, gsem):
        kk = lax.rem(k, CHUNK)
        pltpu.make_async_copy(h2_hbm.at[src_v.at[k]], buf, gsem).wait()

        @pl.loop(0, EB // L)
        def _(g):
            w16 = w_c[kk, pl.ds(g * L, L)]
            for j in range(L):
                sw = jnp.take_along_axis(
                    w16, jnp.full((L,), j, jnp.int32), axis=0)
                e = g * L + j
                for i in range(D // L):
                    sl = pl.ds(i * L, L)
                    buf[e, sl] = buf[e, sl] * sw

        pltpu.sync_copy(buf, acc_s.at[dst_c.at[kk]], add=True)

    pltpu.async_copy(h2_hbm.at[src_v.at[0]], rows0, g0)
    pltpu.async_copy(h2_hbm.at[src_v.at[1]], rows1, g1)

    @pl.loop(0, NBLK, step=2)
    def _(k):
        @pl.when(lax.rem(k, CHUNK) == 0)
        def _():
            kc = pl.multiple_of(wid * NBLK + k, CHUNK)
            pltpu.sync_copy(ei_hbm.at[1, pl.ds(kc, CHUNK)], dst_c)
            pltpu.sync_copy(w_hbm.at[pl.ds(kc, CHUNK)], w_c)

        process(k, rows0, g0)

        @pl.when(k + 2 < NBLK)
        def _():
            pltpu.async_copy(h2_hbm.at[src_v.at[k + 2]], rows0, g0)

        process(k + 1, rows1, g1)

        @pl.when(k + 3 < NBLK)
        def _():
            pltpu.async_copy(h2_hbm.at[src_v.at[k + 3]], rows1, g1)

    plsc.subcore_barrier()

    def _write_copy(row, nrows):
        pltpu.sync_copy(acc_s.at[pl.ds(row, nrows)],
                        out_hbm.at[c, pl.ds(row, nrows)])

    _stripe_copy(_write_copy)


def _dis_from(dp_ref):
    deg = dp_ref[0, 0, :] + dp_ref[1, 0, :]
    return jnp.where(deg > 0, 1.0 / jnp.sqrt(jnp.maximum(deg, 1e-12)), 0.0)


_DEG_SPEC = pl.BlockSpec((NC, 1, MB), lambda i: (0, 0, i))


def _mm_body(x_ref, w_ref, o_ref):
    o_ref[...] = jnp.dot(x_ref[...], w_ref[...],
                         preferred_element_type=jnp.float32)


_mm_call = pl.pallas_call(
    _mm_body,
    grid=(GRID,),
    in_specs=[
        pl.BlockSpec((MB, D), lambda i: (i, 0)),
        pl.BlockSpec((D, D), lambda i: (0, 0)),
    ],
    out_specs=pl.BlockSpec((MB, D), lambda i: (i, 0)),
    out_shape=jax.ShapeDtypeStruct((NPAD, D), jnp.float32),
)


def _h2_body(h_ref, dp_ref, o_ref):
    dis = _dis_from(dp_ref)
    o_ref[...] = h_ref[...] * dis[:, None]


_h2_call = pl.pallas_call(
    _h2_body,
    grid=(GRID,),
    in_specs=[
        pl.BlockSpec((MB, D), lambda i: (i, 0)),
        _DEG_SPEC,
    ],
    out_specs=pl.BlockSpec((MB, D), lambda i: (i, 0)),
    out_shape=jax.ShapeDtypeStruct((NPAD, D), jnp.float32),
)


def _fin_body(p_ref, dp_ref, b_ref, o_ref):
    dis = _dis_from(dp_ref)
    z = (p_ref[0] + p_ref[1]) * dis[:, None] + b_ref[...]
    o_ref[...] = jnp.where(z > 0, z, 0.1 * (jnp.exp(z) - 1.0))


_fin_call = pl.pallas_call(
    _fin_body,
    grid=(GRID,),
    in_specs=[
        pl.BlockSpec((NC, MB, D), lambda i: (0, i, 0)),
        _DEG_SPEC,
        pl.BlockSpec((D,), lambda i: (0,)),
    ],
    out_specs=pl.BlockSpec((MB, D), lambda i: (i, 0)),
    out_shape=jax.ShapeDtypeStruct((NPAD, D), jnp.float32),
)


def kernel(x, edge_index, edge_attrs, W, b):
    pad = E_PAD - E
    pad_idx = jnp.arange(pad, dtype=jnp.int32) & 8191
    ei_p = jnp.concatenate(
        [edge_index.reshape(2, E // EB, EB),
         jnp.broadcast_to(pad_idx.reshape(1, pad // EB, EB),
                          (2, pad // EB, EB))], axis=1)
    w_p = jnp.concatenate(
        [edge_attrs, jnp.zeros((pad,), jnp.float32)]).reshape(NW * NBLK, EB)

    x_pad = jnp.concatenate(
        [x, jnp.zeros((NPAD - N, D), jnp.float32)], axis=0)
    h_raw = _mm_call(x_pad, W)
    degp = _deg_kernel(ei_p, w_p, jnp.zeros((DEG_STRIPE,), jnp.float32))
    h2 = _h2_call(h_raw, degp)
    outp = _agg_kernel(h2, ei_p, w_p,
                       jnp.zeros((EB, D), jnp.float32))
    return _fin_call(outp, degp, b)[:N]


# speedup vs baseline: 1.1901x; 1.0282x over previous
import functools

import jax
import jax.numpy as jnp
from jax import lax
from jax.experimental import pallas as pl
from jax.experimental.pallas import tpu as pltpu
from jax.experimental.pallas import tpu_sc as plsc

N = 10000
D = 128
E = 320000

NC = 2
NS = 16
NW = NC * NS
L = 16

EB = 128
NBLK = 80
CHUNK = 16
BPT = NBLK * EB
E_PAD = NW * BPT
NPAD = 10240
DEG_STRIPE = NPAD // NS
ROW_STRIPE = NPAD // NS
ZROWS = 16

MB = 2048
GRID = NPAD // MB

_MESH = plsc.VectorSubcoreMesh(core_axis_name="c", subcore_axis_name="s")


@functools.partial(
    pl.kernel,
    out_type=jax.ShapeDtypeStruct((NC, 1, NPAD), jnp.float32),
    mesh=_MESH,
    scratch_types=[
        pltpu.VMEM((NBLK, EB), jnp.int32),
        pltpu.VMEM((NBLK, EB), jnp.float32),
        pltpu.VMEM((DEG_STRIPE,), jnp.float32),
        pltpu.VMEM_SHARED((NPAD,), jnp.float32),
        pltpu.SemaphoreType.DMA,
    ],
)
def _deg_kernel(ei_hbm, w_hbm, zero_hbm, out_hbm, idx_v, w_v, z_v, acc_s, sem):
    c = lax.axis_index("c")
    s = lax.axis_index("s")
    wid = c * NS + s
    pltpu.sync_copy(zero_hbm, z_v)
    pltpu.sync_copy(z_v, acc_s.at[pl.ds(s * DEG_STRIPE, DEG_STRIPE)])
    plsc.subcore_barrier()
    pltpu.sync_copy(ei_hbm.at[1, pl.ds(wid * NBLK, NBLK)], idx_v)
    pltpu.sync_copy(w_hbm.at[pl.ds(wid * NBLK, NBLK)], w_v)

    @pl.loop(0, NBLK, step=8)
    def _(k):
        for j in range(8):
            pltpu.async_copy(w_v.at[k + j], acc_s.at[idx_v.at[k + j]], sem,
                             add=True)
        for j in range(8):
            pltpu.make_async_copy(w_v.at[k + j], acc_s.at[idx_v.at[k + j]],
                                  sem).wait()

    plsc.subcore_barrier()
    pltpu.sync_copy(acc_s.at[pl.ds(s * DEG_STRIPE, DEG_STRIPE)],
                    out_hbm.at[c, 0, pl.ds(s * DEG_STRIPE, DEG_STRIPE)])


@functools.partial(
    pl.kernel,
    out_type=jax.ShapeDtypeStruct((NC, NPAD, D), jnp.float32),
    mesh=_MESH,
    scratch_types=[
        pltpu.VMEM((NBLK, EB), jnp.int32),
        pltpu.VMEM((CHUNK, EB), jnp.int32),
        pltpu.VMEM((CHUNK, EB), jnp.float32),
        pltpu.VMEM((EB, D), jnp.float32),
        pltpu.VMEM((EB, D), jnp.float32),
        pltpu.VMEM_SHARED((N, D), jnp.float32),
        pltpu.SemaphoreType.DMA,
        pltpu.SemaphoreType.DMA,
    ],
)
def _agg_kernel(h2_hbm, ei_hbm, w_hbm, zero_hbm, out_hbm,
                src_v, dst_c, w_c, rows0, rows1, acc_s, g0, g1):
    c = lax.axis_index("c")
    s = lax.axis_index("s")
    wid = c * NS + s
    pltpu.sync_copy(zero_hbm, rows0)

    def _stripe_copy(copy_fn):
        base = s * ROW_STRIPE

        @pl.when(s < NS - 1)
        def _():
            @pl.loop(0, ROW_STRIPE // EB)
            def _(r):
                copy_fn(base + r * EB, EB)

        @pl.when(s == NS - 1)
        def _():
            @pl.loop(0, 3)
            def _(r):
                copy_fn(base + r * EB, EB)

            copy_fn(base + 3 * EB, ZROWS)

    def _zero_copy(row, nrows):
        pltpu.sync_copy(rows0.at[pl.ds(0, nrows)], acc_s.at[pl.ds(row, nrows)])

    _stripe_copy(_zero_copy)
    pltpu.sync_copy(ei_hbm.at[0, pl.ds(wid * NBLK, NBLK)], src_v)
    plsc.subcore_barrier()

    def process(k, buf, gsem):
        kk = lax.rem(k, CHUNK)
        pltpu.make_async_copy(h2_hbm.at[src_v.at[k]], buf, gsem).wait()

        @pl.loop(0, EB // L)
        def _(g):
            w16 = w_c[kk, pl.ds(g * L, L)]
            for j in range(L):
                sw = jnp.take_along_axis(
                    w16, jnp.full((L,), j, jnp.int32), axis=0)
                e = g * L + j
                for i in range(D // L):
                    sl = pl.ds(i * L, L)
                    buf[e, sl] = buf[e, sl] * sw

        pltpu.sync_copy(buf, acc_s.at[dst_c.at[kk]], add=True)

    pltpu.async_copy(h2_hbm.at[src_v.at[0]], rows0, g0)
    pltpu.async_copy(h2_hbm.at[src_v.at[1]], rows1, g1)

    @pl.loop(0, NBLK, step=2)
    def _(k):
        @pl.when(lax.rem(k, CHUNK) == 0)
        def _():
            kc = pl.multiple_of(wid * NBLK + k, CHUNK)
            pltpu.sync_copy(ei_hbm.at[1, pl.ds(kc, CHUNK)], dst_c)
            pltpu.sync_copy(w_hbm.at[pl.ds(kc, CHUNK)], w_c)

        process(k, rows0, g0)

        @pl.when(k + 2 < NBLK)
        def _():
            pltpu.async_copy(h2_hbm.at[src_v.at[k + 2]], rows0, g0)

        process(k + 1, rows1, g1)

        @pl.when(k + 3 < NBLK)
        def _():
            pltpu.async_copy(h2_hbm.at[src_v.at[k + 3]], rows1, g1)

    plsc.subcore_barrier()

    def _write_copy(row, nrows):
        pltpu.sync_copy(acc_s.at[pl.ds(row, nrows)],
                        out_hbm.at[c, pl.ds(row, nrows)])

    _stripe_copy(_write_copy)


def _dis_from(dp_ref):
    deg = dp_ref[0, 0, :] + dp_ref[1, 0, :]
    return jnp.where(deg > 0, 1.0 / jnp.sqrt(jnp.maximum(deg, 1e-12)), 0.0)


_DEG_SPEC = pl.BlockSpec((NC, 1, MB), lambda i: (0, 0, i))


def _mm_body(x_ref, w_ref, o_ref):
    o_ref[...] = jnp.dot(x_ref[...], w_ref[...],
                         preferred_element_type=jnp.float32)


_mm_call = pl.pallas_call(
    _mm_body,
    grid=(GRID,),
    in_specs=[
        pl.BlockSpec((MB, D), lambda i: (i, 0)),
        pl.BlockSpec((D, D), lambda i: (0, 0)),
    ],
    out_specs=pl.BlockSpec((MB, D), lambda i: (i, 0)),
    out_shape=jax.ShapeDtypeStruct((NPAD, D), jnp.float32),
)


def _h2_body(h_ref, dp_ref, o_ref):
    dis = _dis_from(dp_ref)
    o_ref[...] = h_ref[...] * dis[:, None]


_h2_call = pl.pallas_call(
    _h2_body,
    grid=(GRID,),
    in_specs=[
        pl.BlockSpec((MB, D), lambda i: (i, 0)),
        _DEG_SPEC,
    ],
    out_specs=pl.BlockSpec((MB, D), lambda i: (i, 0)),
    out_shape=jax.ShapeDtypeStruct((NPAD, D), jnp.float32),
)


def _fin_body(p_ref, dp_ref, b_ref, o_ref):
    dis = _dis_from(dp_ref)
    z = (p_ref[0] + p_ref[1]) * dis[:, None] + b_ref[...]
    o_ref[...] = jnp.where(z > 0, z, 0.1 * (jnp.exp(z) - 1.0))


_fin_call = pl.pallas_call(
    _fin_body,
    grid=(GRID,),
    in_specs=[
        pl.BlockSpec((NC, MB, D), lambda i: (0, i, 0)),
        _DEG_SPEC,
        pl.BlockSpec((D,), lambda i: (0,)),
    ],
    out_specs=pl.BlockSpec((MB, D), lambda i: (i, 0)),
    out_shape=jax.ShapeDtypeStruct((N, D), jnp.float32),
)


def kernel(x, edge_index, edge_attrs, W, b):
    pad = E_PAD - E
    pad_idx = jnp.arange(pad, dtype=jnp.int32) & 8191
    ei_p = jnp.concatenate(
        [edge_index.reshape(2, E // EB, EB),
         jnp.broadcast_to(pad_idx.reshape(1, pad // EB, EB),
                          (2, pad // EB, EB))], axis=1)
    w_p = jnp.concatenate(
        [edge_attrs, jnp.zeros((pad,), jnp.float32)]).reshape(NW * NBLK, EB)

    h_raw = _mm_call(x, W)
    degp = _deg_kernel(ei_p, w_p, jnp.zeros((DEG_STRIPE,), jnp.float32))
    h2 = _h2_call(h_raw, degp)
    outp = _agg_kernel(h2, ei_p, w_p,
                       jnp.zeros((EB, D), jnp.float32))
    return _fin_call(outp, degp, b)
